# Initial kernel scaffold; baseline (speedup 1.0000x reference)
import jax
import jax.numpy as jnp
from jax.experimental import pallas as pl


def kernel(x, emb, W1, b1, W2, b2, W3, b3):
    raise NotImplementedError("write your pallas kernel here")


# speedup vs baseline: 2.6498x; 2.6498x over previous
import functools

import jax
import jax.numpy as jnp
from jax import lax
from jax.experimental import pallas as pl
from jax.experimental.pallas import tpu as pltpu
from jax.experimental.pallas import tpu_sc as plsc

NC = 2
NS = 16
NW = NC * NS
LANES = 16


@functools.lru_cache(maxsize=None)
def _make_pool(B, L, E, interpret=False):
    assert B % NW == 0
    bpw = B // NW
    ecols = E // LANES

    mesh = plsc.VectorSubcoreMesh(
        core_axis_name="c", subcore_axis_name="s", num_cores=NC, num_subcores=NS)

    @functools.partial(
        pl.kernel,
        out_type=jax.ShapeDtypeStruct((B, E), jnp.float32),
        mesh=mesh,
        scratch_types=[
            pltpu.VMEM((L,), jnp.int32),
            pltpu.VMEM((L,), jnp.int32),
            pltpu.VMEM((2, L, E), jnp.float32),
            pltpu.VMEM((bpw, E), jnp.float32),
            pltpu.SemaphoreType.DMA,
            pltpu.SemaphoreType.DMA,
        ],
        compiler_params=pltpu.CompilerParams(use_tc_tiling_on_sc=False),
        interpret=interpret,
    )
    def pool(x_hbm, emb_hbm, out_hbm, idx0, idx1, rows_v, outbuf, sem0, sem1):
        wid = lax.axis_index("s") * NC + lax.axis_index("c")
        base = wid * bpw
        sems = (sem0, sem1)
        idxs = (idx0, idx1)

        def start_row(r, b):
            pltpu.sync_copy(x_hbm.at[base + r], idxs[b])
            pltpu.async_copy(emb_hbm.at[idxs[b]], rows_v.at[b], sems[b])

        def wait_row(b):
            pltpu.make_async_copy(emb_hbm.at[idxs[b]], rows_v.at[b], sems[b]).wait()

        start_row(0, 0)

        @pl.loop(0, bpw, step=2)
        def _rows(r):
            for b in range(2):
                rr = r + b

                @pl.when(rr + 1 < bpw)
                def _():
                    start_row(rr + 1, 1 - b)

                wait_row(b)

                zeros = tuple(jnp.zeros((LANES,), jnp.float32) for _ in range(ecols))

                @pl.loop(0, L, init_carry=zeros, unroll=8)
                def _red(j, carry):
                    return tuple(
                        carry[c] + rows_v[b, j, pl.ds(c * LANES, LANES)]
                        for c in range(ecols))

                acc = _red
                for c in range(ecols):
                    outbuf[rr, pl.ds(c * LANES, LANES)] = acc[c]

        pltpu.sync_copy(outbuf, out_hbm.at[pl.ds(base, bpw)])

    return pool


@functools.lru_cache(maxsize=None)
def _make_mlp(B, E, H2, H, N, inv_l, interpret=False):
    BM = min(B, 2048)
    assert B % BM == 0

    def body(s_ref, w1_ref, b1_ref, w2_ref, b2_ref, w3_ref, b3_ref, o_ref):
        p = s_ref[...] * inv_l
        h = jnp.dot(p, w1_ref[...], preferred_element_type=jnp.float32)
        h = jnp.maximum(h + b1_ref[...], 0.0)
        h = jnp.dot(h, w2_ref[...], preferred_element_type=jnp.float32)
        h = jnp.maximum(h + b2_ref[...], 0.0)
        o = jnp.dot(h, w3_ref[...], preferred_element_type=jnp.float32)
        o_ref[...] = o + b3_ref[...]

    zero = lambda i: (0, 0)
    return pl.pallas_call(
        body,
        grid=(B // BM,),
        in_specs=[
            pl.BlockSpec((BM, E), lambda i: (i, 0)),
            pl.BlockSpec((E, H2), zero),
            pl.BlockSpec((1, H2), zero),
            pl.BlockSpec((H2, H), zero),
            pl.BlockSpec((1, H), zero),
            pl.BlockSpec((H, N), zero),
            pl.BlockSpec((1, N), zero),
        ],
        out_specs=pl.BlockSpec((BM, N), lambda i: (i, 0)),
        out_shape=jax.ShapeDtypeStruct((B, N), jnp.float32),
        interpret=interpret,
    )


def _run(x, emb, W1, b1, W2, b2, W3, b3, interpret=False):
    B, L = x.shape
    V, E = emb.shape
    H2 = W1.shape[1]
    H = W2.shape[1]
    N = W3.shape[1]
    sums = _make_pool(B, L, E, interpret)(x.astype(jnp.int32), emb)
    mlp = _make_mlp(B, E, H2, H, N, 1.0 / L, interpret)
    return mlp(sums, W1, b1.reshape(1, -1), W2, b2.reshape(1, -1),
               W3, b3.reshape(1, -1))


def kernel(x, emb, W1, b1, W2, b2, W3, b3):
    return _run(x, emb, W1, b1, W2, b2, W3, b3)


# speedup vs baseline: 3.4289x; 1.2940x over previous
import functools

import jax
import jax.numpy as jnp
from jax import lax
from jax.experimental import pallas as pl
from jax.experimental.pallas import tpu as pltpu
from jax.experimental.pallas import tpu_sc as plsc

NC = 2
NS = 16
NW = NC * NS
LANES = 16


@functools.lru_cache(maxsize=None)
def _make_pool(B, L, E, interpret=False):
    assert B % NW == 0
    bpw = B // NW
    ecols = E // LANES

    mesh = plsc.VectorSubcoreMesh(
        core_axis_name="c", subcore_axis_name="s", num_cores=NC, num_subcores=NS)

    IBLK = 32
    NBUF = 4
    assert bpw % NBUF == 0 and bpw % IBLK == 0

    @functools.partial(
        pl.kernel,
        out_type=jax.ShapeDtypeStruct((B, E), jnp.float32),
        mesh=mesh,
        scratch_types=[
            pltpu.VMEM((2, IBLK, L), jnp.int32),
            pltpu.VMEM((NBUF, L, E), jnp.float32),
            pltpu.VMEM((bpw, E), jnp.float32),
            pltpu.SemaphoreType.DMA,
            pltpu.SemaphoreType.DMA,
            pltpu.SemaphoreType.DMA,
            pltpu.SemaphoreType.DMA,
        ],
        compiler_params=pltpu.CompilerParams(use_tc_tiling_on_sc=False),
        interpret=interpret,
    )
    def pool(x_hbm, emb_hbm, out_hbm, idxblk, rows_v, outbuf, *sems):
        wid = lax.axis_index("s") * NC + lax.axis_index("c")
        base = wid * bpw

        def load_iblk(r):
            blk = r // IBLK
            pltpu.sync_copy(
                x_hbm.at[pl.ds(base + blk * IBLK, IBLK)], idxblk.at[blk % 2])

        def idx_view(r):
            return idxblk.at[(r // IBLK) % 2, r % IBLK]

        def start_row(r, b):
            pltpu.async_copy(emb_hbm.at[idx_view(r)], rows_v.at[b], sems[b])

        def wait_row(r, b):
            pltpu.make_async_copy(
                emb_hbm.at[idx_view(r)], rows_v.at[b], sems[b]).wait()

        load_iblk(0)
        for j in range(NBUF - 1):
            start_row(j, j)

        @pl.loop(0, bpw, step=NBUF)
        def _rows(r):
            for j in range(NBUF):
                rr = r + j
                nxt = rr + (NBUF - 1)

                @pl.when(jnp.logical_and(nxt % IBLK == 0, nxt < bpw))
                def _():
                    load_iblk(nxt)

                @pl.when(nxt < bpw)
                def _():
                    start_row(nxt, (j + NBUF - 1) % NBUF)

                wait_row(rr, j)

                zeros = tuple(jnp.zeros((LANES,), jnp.float32) for _ in range(ecols))

                @pl.loop(0, L, init_carry=zeros, unroll=8)
                def _red(k, carry):
                    return tuple(
                        carry[c] + rows_v[j, k, pl.ds(c * LANES, LANES)]
                        for c in range(ecols))

                acc = _red
                for c in range(ecols):
                    outbuf[rr, pl.ds(c * LANES, LANES)] = acc[c]

        pltpu.sync_copy(outbuf, out_hbm.at[pl.ds(base, bpw)])

    return pool


@functools.lru_cache(maxsize=None)
def _make_mlp(B, E, H2, H, N, inv_l, interpret=False):
    BM = min(B, 2048)
    assert B % BM == 0

    def body(s_ref, w1_ref, b1_ref, w2_ref, b2_ref, w3_ref, b3_ref, o_ref):
        p = s_ref[...] * inv_l
        h = jnp.dot(p, w1_ref[...], preferred_element_type=jnp.float32)
        h = jnp.maximum(h + b1_ref[...], 0.0)
        h = jnp.dot(h, w2_ref[...], preferred_element_type=jnp.float32)
        h = jnp.maximum(h + b2_ref[...], 0.0)
        o = jnp.dot(h, w3_ref[...], preferred_element_type=jnp.float32)
        o_ref[...] = o + b3_ref[...]

    zero = lambda i: (0, 0)
    return pl.pallas_call(
        body,
        grid=(B // BM,),
        in_specs=[
            pl.BlockSpec((BM, E), lambda i: (i, 0)),
            pl.BlockSpec((E, H2), zero),
            pl.BlockSpec((1, H2), zero),
            pl.BlockSpec((H2, H), zero),
            pl.BlockSpec((1, H), zero),
            pl.BlockSpec((H, N), zero),
            pl.BlockSpec((1, N), zero),
        ],
        out_specs=pl.BlockSpec((BM, N), lambda i: (i, 0)),
        out_shape=jax.ShapeDtypeStruct((B, N), jnp.float32),
        interpret=interpret,
    )


def _run(x, emb, W1, b1, W2, b2, W3, b3, interpret=False):
    B, L = x.shape
    V, E = emb.shape
    H2 = W1.shape[1]
    H = W2.shape[1]
    N = W3.shape[1]
    sums = _make_pool(B, L, E, interpret)(x.astype(jnp.int32), emb)
    mlp = _make_mlp(B, E, H2, H, N, 1.0 / L, interpret)
    return mlp(sums, W1, b1.reshape(1, -1), W2, b2.reshape(1, -1),
               W3, b3.reshape(1, -1))


def kernel(x, emb, W1, b1, W2, b2, W3, b3):
    return _run(x, emb, W1, b1, W2, b2, W3, b3)
